# baseline (device time: 141349 ns/iter reference)
import jax
import jax.numpy as jnp
from jax import lax
from jax.experimental import pallas as pl
from jax.experimental.pallas import tpu as pltpu

N_DEV = 16
N_STREAMS = 12

_R = [0, 1, 5, 9, 13, 14, 10, 6, 2, 3, 7, 11, 15, 12, 8, 4]
_POS = [0] * N_DEV
for _p, _r in enumerate(_R):
    _POS[_r] = _p
_NEXT = [_R[(_POS[r] + 1) % N_DEV] for r in range(N_DEV)]
_PREV = [_R[(_POS[r] - 1) % N_DEV] for r in range(N_DEV)]


def kernel(A, B):
    m, k = A.shape
    _, n = B.shape
    chunk = m // N_DEV
    cblk = n // N_STREAMS

    def body(idx_ref, a_ref, b_ref, out_ref, rs_buf, rs_send_sems,
             ag_send_sems, rs_sems, ag_sems):
        pos = idx_ref[0]
        nxt = idx_ref[1]
        prv = idx_ref[2]

        def send_chunk_idx(t, s):
            if t % 2 == 0:
                return lax.rem(pos - s + N_DEV, N_DEV)
            return lax.rem(pos + s, N_DEV)

        def recv_chunk_idx(t, s):
            if t % 2 == 0:
                return lax.rem(pos - s - 1 + 2 * N_DEV, N_DEV)
            return lax.rem(pos + s + 1, N_DEV)

        def ag_send_chunk_idx(t, s):
            if t % 2 == 0:
                return lax.rem(pos + 1 - s + 2 * N_DEV, N_DEV)
            return lax.rem(pos - 1 + s + N_DEV, N_DEV)

        def nbr(t):
            return nxt if t % 2 == 0 else prv

        sent = []

        def start_rs_send(t, s):
            c = send_chunk_idx(t, s)
            rdma = pltpu.make_async_remote_copy(
                src_ref=out_ref.at[
                    pl.ds(c * chunk, chunk), pl.ds(t * cblk, cblk)
                ],
                dst_ref=rs_buf.at[t, s],
                send_sem=rs_send_sems.at[t, s],
                recv_sem=rs_sems.at[t, s],
                device_id=nbr(t),
                device_id_type=pl.DeviceIdType.LOGICAL,
            )
            rdma.start()
            sent.append(rdma)
            return rdma

        def start_ag_send(t, s):
            c = ag_send_chunk_idx(t, s)
            rdma = pltpu.make_async_remote_copy(
                src_ref=out_ref.at[
                    pl.ds(c * chunk, chunk), pl.ds(t * cblk, cblk)
                ],
                dst_ref=out_ref.at[
                    pl.ds(c * chunk, chunk), pl.ds(t * cblk, cblk)
                ],
                send_sem=ag_send_sems.at[t, s],
                recv_sem=ag_sems.at[t, s],
                device_id=nbr(t),
                device_id_type=pl.DeviceIdType.LOGICAL,
            )
            rdma.start()
            sent.append(rdma)
            return rdma

        rs_rdmas = [[None] * (N_DEV - 1) for _ in range(N_STREAMS)]
        for t in range(N_STREAMS):
            out_ref[:, pl.ds(t * cblk, cblk)] = jnp.dot(
                a_ref[:, :],
                b_ref[:, pl.ds(t * cblk, cblk)],
                preferred_element_type=jnp.float32,
            )
            rs_rdmas[t][0] = start_rs_send(t, 0)

        ag_rdmas = [[None] * (N_DEV - 1) for _ in range(N_STREAMS)]
        for s in range(N_DEV - 1):
            for t in range(N_STREAMS):
                rs_rdmas[t][s].wait_recv()
                c = recv_chunk_idx(t, s)
                out_ref[pl.ds(c * chunk, chunk), pl.ds(t * cblk, cblk)] = (
                    out_ref[pl.ds(c * chunk, chunk), pl.ds(t * cblk, cblk)]
                    + rs_buf[t, s]
                )
                if s < N_DEV - 2:
                    rs_rdmas[t][s + 1] = start_rs_send(t, s + 1)
                else:
                    ag_rdmas[t][0] = start_ag_send(t, 0)

        for s in range(N_DEV - 1):
            for t in range(N_STREAMS):
                ag_rdmas[t][s].wait_recv()
                if s < N_DEV - 2:
                    ag_rdmas[t][s + 1] = start_ag_send(t, s + 1)

        for rdma in sent:
            rdma.wait_send()

        out_ref[:, :] = jnp.maximum(out_ref[:, :], 0.0)

    my = lax.axis_index("i")
    idx = jnp.stack([
        jnp.array(_POS, dtype=jnp.int32)[my],
        jnp.array(_NEXT, dtype=jnp.int32)[my],
        jnp.array(_PREV, dtype=jnp.int32)[my],
    ])

    return pl.pallas_call(
        body,
        out_shape=jax.ShapeDtypeStruct((m, n), jnp.float32),
        in_specs=[
            pl.BlockSpec(memory_space=pltpu.SMEM),
            pl.BlockSpec(memory_space=pltpu.VMEM),
            pl.BlockSpec(memory_space=pltpu.VMEM),
        ],
        out_specs=pl.BlockSpec(memory_space=pltpu.VMEM),
        scratch_shapes=[
            pltpu.VMEM((N_STREAMS, N_DEV - 1, chunk, cblk), jnp.float32),
            pltpu.SemaphoreType.DMA((N_STREAMS, N_DEV - 1)),
            pltpu.SemaphoreType.DMA((N_STREAMS, N_DEV - 1)),
            pltpu.SemaphoreType.DMA((N_STREAMS, N_DEV - 1)),
            pltpu.SemaphoreType.DMA((N_STREAMS, N_DEV - 1)),
        ],
    )(idx, A, B)


# device time: 117005 ns/iter; 1.2081x vs baseline; 1.2081x over previous
import jax
import jax.numpy as jnp
from jax import lax
from jax.experimental import pallas as pl
from jax.experimental.pallas import tpu as pltpu

N_DEV = 16
N_STREAMS = 6

_R = [0, 1, 5, 9, 13, 14, 10, 6, 2, 3, 7, 11, 15, 12, 8, 4]
_POS = [0] * N_DEV
for _p, _r in enumerate(_R):
    _POS[_r] = _p
_NEXT = [_R[(_POS[r] + 1) % N_DEV] for r in range(N_DEV)]
_PREV = [_R[(_POS[r] - 1) % N_DEV] for r in range(N_DEV)]


def kernel(A, B):
    m, k = A.shape
    _, n = B.shape
    rblk = m // N_STREAMS
    chunk = rblk // N_DEV

    def body(tbl_ref, a_ref, b_ref, out_ref, rs_buf, rs_send_sems,
             ag_send_sems, rs_sems, ag_sems):
        my = lax.axis_index("i")
        pos = tbl_ref[0, my]
        nxt = tbl_ref[1, my]
        prv = tbl_ref[2, my]

        barrier_sem = pltpu.get_barrier_semaphore()
        pl.semaphore_signal(barrier_sem, inc=1, device_id=nxt,
                            device_id_type=pl.DeviceIdType.LOGICAL)
        pl.semaphore_signal(barrier_sem, inc=1, device_id=prv,
                            device_id_type=pl.DeviceIdType.LOGICAL)
        pl.semaphore_wait(barrier_sem, 2)

        def send_chunk_idx(t, s):
            if t % 2 == 0:
                return lax.rem(pos - s + N_DEV, N_DEV)
            return lax.rem(pos + s, N_DEV)

        def recv_chunk_idx(t, s):
            if t % 2 == 0:
                return lax.rem(pos - s - 1 + 2 * N_DEV, N_DEV)
            return lax.rem(pos + s + 1, N_DEV)

        def ag_send_chunk_idx(t, s):
            if t % 2 == 0:
                return lax.rem(pos + 1 - s + 2 * N_DEV, N_DEV)
            return lax.rem(pos - 1 + s + N_DEV, N_DEV)

        def nbr(t):
            return nxt if t % 2 == 0 else prv

        sent_by_stream = [[] for _ in range(N_STREAMS)]

        def start_rs_send(t, s):
            c = send_chunk_idx(t, s)
            rdma = pltpu.make_async_remote_copy(
                src_ref=out_ref.at[pl.ds(t * rblk + c * chunk, chunk), :],
                dst_ref=rs_buf.at[t, s],
                send_sem=rs_send_sems.at[t, s],
                recv_sem=rs_sems.at[t, s],
                device_id=nbr(t),
                device_id_type=pl.DeviceIdType.LOGICAL,
            )
            rdma.start()
            sent_by_stream[t].append(rdma)
            return rdma

        def start_ag_send(t, s):
            c = ag_send_chunk_idx(t, s)
            rdma = pltpu.make_async_remote_copy(
                src_ref=out_ref.at[pl.ds(t * rblk + c * chunk, chunk), :],
                dst_ref=out_ref.at[pl.ds(t * rblk + c * chunk, chunk), :],
                send_sem=ag_send_sems.at[t, s],
                recv_sem=ag_sems.at[t, s],
                device_id=nbr(t),
                device_id_type=pl.DeviceIdType.LOGICAL,
            )
            rdma.start()
            sent_by_stream[t].append(rdma)
            return rdma

        rs_rdmas = [[None] * (N_DEV - 1) for _ in range(N_STREAMS)]
        for t in range(N_STREAMS):
            out_ref[pl.ds(t * rblk, rblk), :] = jnp.dot(
                a_ref[pl.ds(t * rblk, rblk), :],
                b_ref[:, :],
                preferred_element_type=jnp.float32,
            )
            rs_rdmas[t][0] = start_rs_send(t, 0)

        ag_rdmas = [[None] * (N_DEV - 1) for _ in range(N_STREAMS)]
        for s in range(N_DEV - 1):
            for t in range(N_STREAMS):
                rs_rdmas[t][s].wait_recv()
                c = recv_chunk_idx(t, s)
                off = t * rblk + c * chunk
                out_ref[pl.ds(off, chunk), :] = (
                    out_ref[pl.ds(off, chunk), :] + rs_buf[t, s]
                )
                if s < N_DEV - 2:
                    rs_rdmas[t][s + 1] = start_rs_send(t, s + 1)
                else:
                    ag_rdmas[t][0] = start_ag_send(t, 0)

        for s in range(N_DEV - 1):
            for t in range(N_STREAMS):
                ag_rdmas[t][s].wait_recv()
                if s < N_DEV - 2:
                    ag_rdmas[t][s + 1] = start_ag_send(t, s + 1)

        for t in range(N_STREAMS):
            for rdma in sent_by_stream[t]:
                rdma.wait_send()
            out_ref[pl.ds(t * rblk, rblk), :] = jnp.maximum(
                out_ref[pl.ds(t * rblk, rblk), :], 0.0
            )

    tbl = jnp.array([_POS, _NEXT, _PREV], dtype=jnp.int32)

    return pl.pallas_call(
        body,
        out_shape=jax.ShapeDtypeStruct((m, n), jnp.float32),
        in_specs=[
            pl.BlockSpec(memory_space=pltpu.SMEM),
            pl.BlockSpec(memory_space=pltpu.VMEM),
            pl.BlockSpec(memory_space=pltpu.VMEM),
        ],
        out_specs=pl.BlockSpec(memory_space=pltpu.VMEM),
        scratch_shapes=[
            pltpu.VMEM((N_STREAMS, N_DEV - 1, chunk, n), jnp.float32),
            pltpu.SemaphoreType.DMA((N_STREAMS, N_DEV - 1)),
            pltpu.SemaphoreType.DMA((N_STREAMS, N_DEV - 1)),
            pltpu.SemaphoreType.DMA((N_STREAMS, N_DEV - 1)),
            pltpu.SemaphoreType.DMA((N_STREAMS, N_DEV - 1)),
        ],
        compiler_params=pltpu.CompilerParams(collective_id=0),
    )(tbl, A, B)
